# baseline (speedup 1.0000x reference)
import functools

import jax
import jax.numpy as jnp
from jax import lax
from jax.experimental import pallas as pl
from jax.experimental.pallas import tpu as pltpu
from jax.experimental.pallas import tpu_sc as plsc

E = 160000
H = 256
NUM_CORES = 2
NUM_SUBCORES = 16
NW = NUM_CORES * NUM_SUBCORES
L = 16
CHUNK = 64
NCHUNKS = E // CHUNK
K = -(-NCHUNKS // NW)
KE = K * CHUNK

_mesh = plsc.VectorSubcoreMesh(core_axis_name="c", subcore_axis_name="s")


@functools.partial(
    pl.kernel,
    out_type=jax.ShapeDtypeStruct((E * H,), jnp.float32),
    mesh=_mesh,
    compiler_params=pltpu.CompilerParams(needs_layout_passes=False),
    scratch_types=[
        pltpu.VMEM((KE,), jnp.int32),
        pltpu.VMEM((4 * H,), jnp.float32),
        pltpu.VMEM((CHUNK * H,), jnp.float32),
        pltpu.VMEM((CHUNK * H,), jnp.float32),
        pltpu.SemaphoreType.DMA,
        pltpu.SemaphoreType.DMA,
    ],
)
def _embed(idx_hbm, w_hbm, out_hbm, idx_v, table_v, bb0, bb1, bw0, bw1):
    bb = (bb0, bb1)
    bwsem = (bw0, bw1)

    wid = lax.axis_index("s") * NUM_CORES + lax.axis_index("c")
    base = wid * K
    nvalid = jnp.minimum(K, NCHUNKS - base)
    start_e = pl.multiple_of(jnp.minimum(base * CHUNK, E - KE), 8)
    loff_e = pl.multiple_of(base * CHUNK - start_e, 8)

    pltpu.sync_copy(idx_hbm.at[pl.ds(start_e, KE)], idx_v)
    pltpu.sync_copy(w_hbm, table_v)

    lanes = lax.iota(jnp.int32, L)
    lane_of = [jnp.full((L,), j, jnp.int32) for j in range(L)]

    def out_slice(t):
        return out_hbm.at[pl.ds(
            pl.multiple_of((base + t) * (CHUNK * H), 8), CHUNK * H)]

    def bw_start(t, b):
        pltpu.make_async_copy(bb[b], out_slice(t), bwsem[b]).start()

    def bw_wait(t, b):
        pltpu.make_async_copy(bb[b], out_slice(t), bwsem[b]).wait()

    def build_chunk(t, buf):
        for rg in range(CHUNK // L):
            pos16 = pl.multiple_of(loff_e + t * CHUNK + rg * L, 8)
            iv16 = idx_v[pl.ds(pos16, L)]
            for j in range(L):
                ivj = iv16.at[lane_of[j]].get(
                    mode="promise_in_bounds")
                srcv = ivj * H + lanes
                robase = (rg * L + j) * H
                for cg in range(H // L):
                    v = plsc.load_gather(table_v, [srcv + cg * L])
                    buf[pl.ds(robase + cg * L, L)] = v

    def body(i, carry):
        for b in range(2):
            t = i * 2 + b
            prev = t - 2

            @pl.when(t < nvalid)
            def _():
                @pl.when(prev >= 0)
                def _():
                    bw_wait(prev, b)
                build_chunk(t, bb[b])
                bw_start(t, b)

        return carry

    lax.fori_loop(0, (K + 1) // 2, body, 0)

    @pl.when(nvalid > 0)
    def _():
        for p in range(2):
            bw_wait(0, p)


def kernel(edge_attr, weight):
    flat = _embed(edge_attr.astype(jnp.int32),
                  weight.astype(jnp.float32).reshape(-1))
    return flat.reshape(E, H)


# speedup vs baseline: 1.9273x; 1.9273x over previous
import functools

import jax
import jax.numpy as jnp
from jax import lax
from jax.experimental import pallas as pl
from jax.experimental.pallas import tpu as pltpu
from jax.experimental.pallas import tpu_sc as plsc

E = 160000
H = 256
NUM_CORES = 2
NUM_SUBCORES = 16
NW = NUM_CORES * NUM_SUBCORES
L = 16
CHUNK = 64
NCHUNKS = E // CHUNK
K = -(-NCHUNKS // NW)
KE = K * CHUNK

_mesh = plsc.VectorSubcoreMesh(core_axis_name="c", subcore_axis_name="s")


@functools.partial(
    pl.kernel,
    out_type=jax.ShapeDtypeStruct((E, H), jnp.float32),
    mesh=_mesh,
    compiler_params=pltpu.CompilerParams(needs_layout_passes=False),
    scratch_types=[
        pltpu.VMEM((KE,), jnp.int32),
        pltpu.VMEM((4, H), jnp.float32),
        pltpu.VMEM((CHUNK, H), jnp.float32),
        pltpu.VMEM((CHUNK, H), jnp.float32),
        pltpu.SemaphoreType.DMA,
        pltpu.SemaphoreType.DMA,
    ],
)
def _embed(idx_hbm, w_hbm, out_hbm, idx_v, table_v, bb0, bb1, bw0, bw1):
    bb = (bb0, bb1)
    bwsem = (bw0, bw1)

    wid = lax.axis_index("s") * NUM_CORES + lax.axis_index("c")
    base = wid * K
    nvalid = jnp.minimum(K, NCHUNKS - base)
    start_e = pl.multiple_of(jnp.minimum(base * CHUNK, E - KE), 8)
    loff_e = pl.multiple_of(base * CHUNK - start_e, 8)

    pltpu.sync_copy(idx_hbm.at[pl.ds(start_e, KE)], idx_v)
    pltpu.sync_copy(w_hbm, table_v)

    lanes = lax.iota(jnp.int32, L)
    lane_of = [jnp.full((L,), j, jnp.int32) for j in range(L)]

    def out_slice(t):
        return out_hbm.at[pl.ds(pl.multiple_of((base + t) * CHUNK, 8), CHUNK)]

    def bw_start(t, b):
        pltpu.make_async_copy(bb[b], out_slice(t), bwsem[b]).start()

    def bw_wait(t, b):
        pltpu.make_async_copy(bb[b], out_slice(t), bwsem[b]).wait()

    def build_chunk(t, buf):
        def rg_body(rg, carry):
            pos16 = pl.multiple_of(loff_e + t * CHUNK + rg * L, 8)
            iv16 = idx_v[pl.ds(pos16, L)]
            for j in range(L):
                ivj = iv16.at[lane_of[j]].get(
                    mode="promise_in_bounds")
                rowv = jnp.full((L,), rg * L + j, jnp.int32)
                for cg in range(H // L):
                    colv = lanes + cg * L
                    v = plsc.load_gather(table_v, [ivj, colv])
                    plsc.store_scatter(buf, [rowv, colv], v)
            return carry

        lax.fori_loop(0, CHUNK // L, rg_body, 0)

    def body(i, carry):
        for b in range(2):
            t = i * 2 + b
            prev = t - 2

            @pl.when(t < nvalid)
            def _():
                @pl.when(prev >= 0)
                def _():
                    bw_wait(prev, b)
                build_chunk(t, bb[b])
                bw_start(t, b)

        return carry

    lax.fori_loop(0, (K + 1) // 2, body, 0)

    @pl.when(nvalid > 0)
    def _():
        for p in range(2):
            bw_wait(0, p)


def kernel(edge_attr, weight):
    return _embed(edge_attr.astype(jnp.int32), weight.astype(jnp.float32))
